# Initial kernel scaffold; baseline (speedup 1.0000x reference)
import jax
import jax.numpy as jnp
from jax.experimental import pallas as pl


def kernel(enc, dec, codebook, W_pre, b_pre, W_post, b_post):
    raise NotImplementedError("write your pallas kernel here")


# speedup vs baseline: 1.2145x; 1.2145x over previous
import jax
import jax.numpy as jnp
from jax.experimental import pallas as pl

B, N, D, K = 4, 4096, 64, 1024
TILE_N = 512
NT = N // TILE_N


def _fused_kernel(enc_ref, dec_ref, cb_ref, cbt_ref, wpre_ref, bpre_ref,
                  wpost_ref, bpost_ref, out_ref, loss_ref, kl_ref,
                  counts_ref):
    n = pl.program_id(1)

    enc = enc_ref[0]
    dec = dec_ref[0]
    cb = cb_ref[...]
    cbt = cbt_ref[...]

    res = jnp.dot(enc - dec, wpre_ref[...],
                  preferred_element_type=jnp.float32) + bpre_ref[...]

    cb_norm = jnp.sum(cbt * cbt, axis=0).reshape(1, K)
    scores = jnp.dot(res, cbt, preferred_element_type=jnp.float32)
    d2 = cb_norm - 2.0 * scores

    idx = jnp.argmin(d2, axis=1).reshape(TILE_N, 1)
    onehot = (jax.lax.broadcasted_iota(jnp.int32, (TILE_N, K), 1)
              == idx).astype(jnp.float32)

    quant = jnp.dot(onehot, cb, preferred_element_type=jnp.float32)

    part_counts = jnp.sum(onehot, axis=0).reshape(1, 1, K)
    diff = quant - res
    part_loss = jnp.sum(diff * diff).reshape(1, 1, 1)

    @pl.when(n == 0)
    def _init():
        counts_ref[...] = part_counts
        loss_ref[...] = part_loss

    @pl.when(n != 0)
    def _acc():
        counts_ref[...] = counts_ref[...] + part_counts
        loss_ref[...] = loss_ref[...] + part_loss

    @pl.when(n == NT - 1)
    def _finalize():
        loss_ref[...] = loss_ref[...] * (1.0 / (N * D))
        probs = counts_ref[...] * (1.0 / N)
        kl = jnp.sum(probs * jnp.log(probs * K + 1e-10))
        kl_ref[...] = kl.reshape(1, 1, 1)

    out_ref[0] = jnp.dot(quant + dec, wpost_ref[...],
                         preferred_element_type=jnp.float32) + bpost_ref[...]


@jax.jit
def kernel(enc, dec, codebook, W_pre, b_pre, W_post, b_post):
    out, loss, kl, _ = pl.pallas_call(
        _fused_kernel,
        grid=(B, NT),
        in_specs=[
            pl.BlockSpec((1, TILE_N, D), lambda b, n: (b, n, 0)),
            pl.BlockSpec((1, TILE_N, D), lambda b, n: (b, n, 0)),
            pl.BlockSpec((K, D), lambda b, n: (0, 0)),
            pl.BlockSpec((D, K), lambda b, n: (0, 0)),
            pl.BlockSpec((D, D), lambda b, n: (0, 0)),
            pl.BlockSpec((1, D), lambda b, n: (0, 0)),
            pl.BlockSpec((D, D), lambda b, n: (0, 0)),
            pl.BlockSpec((1, D), lambda b, n: (0, 0)),
        ],
        out_specs=[
            pl.BlockSpec((1, TILE_N, D), lambda b, n: (b, n, 0)),
            pl.BlockSpec((1, 1, 1), lambda b, n: (b, 0, 0)),
            pl.BlockSpec((1, 1, 1), lambda b, n: (b, 0, 0)),
            pl.BlockSpec((1, 1, K), lambda b, n: (b, 0, 0)),
        ],
        out_shape=[
            jax.ShapeDtypeStruct((B, N, D), jnp.float32),
            jax.ShapeDtypeStruct((B, 1, 1), jnp.float32),
            jax.ShapeDtypeStruct((B, 1, 1), jnp.float32),
            jax.ShapeDtypeStruct((B, 1, K), jnp.float32),
        ],
    )(enc, dec, codebook, codebook.T, W_pre, b_pre.reshape(1, D), W_post,
      b_post.reshape(1, D))
    loss = loss.reshape(B)
    kl = kl.reshape(B)
    return out, loss, loss, kl


# speedup vs baseline: 1.7155x; 1.4125x over previous
import jax
import jax.numpy as jnp
from jax.experimental import pallas as pl

B, N, D, K = 4, 4096, 64, 1024
TILE_N = 4096
NT = N // TILE_N


def _fused_kernel(enc_ref, dec_ref, cb_ref, cbt_ref, wpre_ref, bpre_ref,
                  wpost_ref, bpost_ref, out_ref, loss_ref, kl_ref,
                  counts_ref):
    n = pl.program_id(1)

    enc = enc_ref[0]
    dec = dec_ref[0]
    cb = cb_ref[...]
    cbt = cbt_ref[...]

    res = jnp.dot(enc - dec, wpre_ref[...],
                  preferred_element_type=jnp.float32) + bpre_ref[...]

    cb_norm = jnp.sum(cbt * cbt, axis=0).reshape(1, K)
    scores = jnp.dot(res, cbt, preferred_element_type=jnp.float32)
    d2 = cb_norm - 2.0 * scores

    idx = jnp.argmin(d2, axis=1).reshape(TILE_N, 1)
    onehot = (jax.lax.broadcasted_iota(jnp.int32, (TILE_N, K), 1)
              == idx).astype(jnp.float32)

    quant = jnp.dot(onehot, cb, preferred_element_type=jnp.float32)

    part_counts = jnp.sum(onehot, axis=0).reshape(1, 1, K)
    diff = quant - res
    part_loss = jnp.sum(diff * diff).reshape(1, 1, 1)

    @pl.when(n == 0)
    def _init():
        counts_ref[...] = part_counts
        loss_ref[...] = part_loss

    @pl.when(n != 0)
    def _acc():
        counts_ref[...] = counts_ref[...] + part_counts
        loss_ref[...] = loss_ref[...] + part_loss

    @pl.when(n == NT - 1)
    def _finalize():
        loss_ref[...] = loss_ref[...] * (1.0 / (N * D))
        probs = counts_ref[...] * (1.0 / N)
        kl = jnp.sum(probs * jnp.log(probs * K + 1e-10))
        kl_ref[...] = kl.reshape(1, 1, 1)

    out_ref[0] = jnp.dot(quant + dec, wpost_ref[...],
                         preferred_element_type=jnp.float32) + bpost_ref[...]


@jax.jit
def kernel(enc, dec, codebook, W_pre, b_pre, W_post, b_post):
    out, loss, kl, _ = pl.pallas_call(
        _fused_kernel,
        grid=(B, NT),
        in_specs=[
            pl.BlockSpec((1, TILE_N, D), lambda b, n: (b, n, 0)),
            pl.BlockSpec((1, TILE_N, D), lambda b, n: (b, n, 0)),
            pl.BlockSpec((K, D), lambda b, n: (0, 0)),
            pl.BlockSpec((D, K), lambda b, n: (0, 0)),
            pl.BlockSpec((D, D), lambda b, n: (0, 0)),
            pl.BlockSpec((1, D), lambda b, n: (0, 0)),
            pl.BlockSpec((D, D), lambda b, n: (0, 0)),
            pl.BlockSpec((1, D), lambda b, n: (0, 0)),
        ],
        out_specs=[
            pl.BlockSpec((1, TILE_N, D), lambda b, n: (b, n, 0)),
            pl.BlockSpec((1, 1, 1), lambda b, n: (b, 0, 0)),
            pl.BlockSpec((1, 1, 1), lambda b, n: (b, 0, 0)),
            pl.BlockSpec((1, 1, K), lambda b, n: (b, 0, 0)),
        ],
        out_shape=[
            jax.ShapeDtypeStruct((B, N, D), jnp.float32),
            jax.ShapeDtypeStruct((B, 1, 1), jnp.float32),
            jax.ShapeDtypeStruct((B, 1, 1), jnp.float32),
            jax.ShapeDtypeStruct((B, 1, K), jnp.float32),
        ],
    )(enc, dec, codebook, codebook.T, W_pre, b_pre.reshape(1, D), W_post,
      b_post.reshape(1, D))
    loss = loss.reshape(B)
    kl = kl.reshape(B)
    return out, loss, loss, kl
